# baseline (speedup 1.0000x reference)
import functools

import jax
import jax.numpy as jnp
from jax import lax
from jax.experimental import pallas as pl
from jax.experimental.pallas import tpu as pltpu
from jax.experimental.pallas import tpu_sc as plsc

_NC = 2
_NS = 16
_BUF_WORDS = 130816


def _chunk_sizes(total):
    sizes = []
    left = total
    while left > 0:
        c = min(left, _BUF_WORDS)
        sizes.append(c)
        left -= c
    return sizes


def kernel(x, pos_table):
    batch, seq_len = x.shape[0], x.shape[1]
    d_model = pos_table.shape[1]
    nw = _NC * _NS
    words = seq_len * d_model
    words_per_w = words // nw
    sizes = _chunk_sizes(words_per_w)

    table_flat = pos_table[:seq_len].reshape(words)

    mesh = plsc.VectorSubcoreMesh(
        core_axis_name="c",
        subcore_axis_name="s",
        num_cores=_NC,
        num_subcores=_NS,
    )

    @functools.partial(
        pl.kernel,
        out_type=jax.ShapeDtypeStruct((batch, words), jnp.float32),
        mesh=mesh,
        scratch_types=[
            pltpu.VMEM((_BUF_WORDS,), jnp.float32),
            pltpu.SemaphoreType.DMA,
        ],
    )
    def broadcast_span(table_hbm, out_hbm, buf, rsem):
        wid = lax.axis_index("s") * _NC + lax.axis_index("c")
        base = wid * words_per_w

        off = 0
        for c in sizes:
            o0 = base + off
            pltpu.async_copy(table_hbm.at[pl.ds(o0, c)], buf.at[pl.ds(0, c)],
                             rsem).wait()
            for b in range(batch):
                pltpu.sync_copy(buf.at[pl.ds(0, c)], out_hbm.at[b, pl.ds(o0, c)])
            off += c

    out = broadcast_span(table_flat)
    return out.reshape(batch, seq_len, d_model)


# speedup vs baseline: 2.7439x; 2.7439x over previous
import functools

import jax
import jax.numpy as jnp
from jax import lax
from jax.experimental import pallas as pl
from jax.experimental.pallas import tpu as pltpu
from jax.experimental.pallas import tpu_sc as plsc

_NC = 2
_NS = 16


def _chunk_sizes(total_rows, max_rows):
    sizes = []
    left = total_rows
    while left > 0:
        c = min(left, max_rows)
        sizes.append(c)
        left -= c
    return sizes


def kernel(x, pos_table):
    batch, seq_len = x.shape[0], x.shape[1]
    d_model = pos_table.shape[1]
    nw = _NC * _NS
    rows_per_w = seq_len // nw
    max_rows = min(rows_per_w, (131064 // d_model) // 8 * 8)
    sizes = _chunk_sizes(rows_per_w, max_rows)

    mesh = plsc.VectorSubcoreMesh(
        core_axis_name="c",
        subcore_axis_name="s",
        num_cores=_NC,
        num_subcores=_NS,
    )

    @functools.partial(
        pl.kernel,
        out_type=jax.ShapeDtypeStruct((batch, seq_len, d_model), jnp.float32),
        mesh=mesh,
        scratch_types=[
            pltpu.VMEM((max_rows, d_model), jnp.float32),
            pltpu.SemaphoreType.DMA,
        ],
    )
    def broadcast_rows(table_hbm, out_hbm, buf, rsem):
        wid = lax.axis_index("s") * _NC + lax.axis_index("c")
        base = wid * rows_per_w

        off = 0
        for c in sizes:
            r0 = base + off
            pltpu.async_copy(table_hbm.at[pl.ds(r0, c)],
                             buf.at[pl.ds(0, c)], rsem).wait()
            for b in range(batch):
                pltpu.sync_copy(buf.at[pl.ds(0, c)],
                                out_hbm.at[b, pl.ds(r0, c)])
            off += c

    return broadcast_rows(pos_table)


# speedup vs baseline: 2.7488x; 1.0018x over previous
import functools

import jax
import jax.numpy as jnp
from jax import lax
from jax.experimental import pallas as pl
from jax.experimental.pallas import tpu as pltpu
from jax.experimental.pallas import tpu_sc as plsc

_NC = 2
_NS = 16


def _chunk_sizes(total_rows, max_rows):
    n = -(-total_rows // max_rows)
    sizes = []
    left = total_rows
    for i in range(n, 0, -1):
        even = (-(-left // i) + 7) // 8 * 8
        c = min(max_rows, even, left)
        sizes.append(c)
        left -= c
    return sizes


def kernel(x, pos_table):
    batch, seq_len = x.shape[0], x.shape[1]
    d_model = pos_table.shape[1]
    nw = _NC * _NS
    rows_per_w = seq_len // nw
    max_rows = min(rows_per_w, (131064 // d_model) // 8 * 8)
    sizes = _chunk_sizes(rows_per_w, max_rows)

    mesh = plsc.VectorSubcoreMesh(
        core_axis_name="c",
        subcore_axis_name="s",
        num_cores=_NC,
        num_subcores=_NS,
    )

    @functools.partial(
        pl.kernel,
        out_type=jax.ShapeDtypeStruct((batch, seq_len, d_model), jnp.float32),
        mesh=mesh,
        scratch_types=[
            pltpu.VMEM((max_rows, d_model), jnp.float32),
            pltpu.SemaphoreType.DMA,
        ],
    )
    def broadcast_rows(table_hbm, out_hbm, buf, rsem):
        wid = lax.axis_index("s") * _NC + lax.axis_index("c")
        base = wid * rows_per_w

        off = 0
        for c in sizes:
            r0 = base + off
            pltpu.async_copy(table_hbm.at[pl.ds(r0, c)],
                             buf.at[pl.ds(0, c)], rsem).wait()
            for b in range(batch):
                pltpu.sync_copy(buf.at[pl.ds(0, c)],
                                out_hbm.at[b, pl.ds(r0, c)])
            off += c

    return broadcast_rows(pos_table)
